# baseline (speedup 1.0000x reference)
import jax
import jax.numpy as jnp
from jax.experimental import pallas as pl


_BB = 16


def _labels_kernel(x_ref, cent_ref, out_ref):
    cent = cent_ref[...]
    cneg = cent * -2.0
    csq = jnp.sum(cent * cent, axis=0)[:, None]
    for i in range(_BB):
        xb = x_ref[i]
        xyneg = jax.lax.dot_general(
            cneg, xb, (((0,), (0,)), ((), ())),
            preferred_element_type=jnp.float32)
        xsq = jnp.sum(xb * xb, axis=0, keepdims=True)
        dist = (xsq + xyneg) + csq
        out_ref[i] = jnp.argmin(dist, axis=0).astype(jnp.int32)[None, :]


def _run(x, centroids):
    b, c, s = x.shape
    k = centroids.shape[1]
    return pl.pallas_call(
        _labels_kernel,
        grid=(b // _BB,),
        in_specs=[
            pl.BlockSpec((_BB, c, s), lambda i: (i, 0, 0)),
            pl.BlockSpec((c, k), lambda i: (0, 0)),
        ],
        out_specs=pl.BlockSpec((_BB, 1, s), lambda i: (i, 0, 0)),
        out_shape=jax.ShapeDtypeStruct((b, 1, s), jnp.int32),
    )(x, centroids)


def kernel(x, centroids):
    b, c, s = x.shape
    return _run(x, centroids).reshape(b, s)
